# baseline (speedup 1.0000x reference)
import functools

import jax
import jax.numpy as jnp
from jax import lax
from jax.experimental import pallas as pl
from jax.experimental.pallas import tpu as pltpu
from jax.experimental.pallas import tpu_sc as plsc

NC, NS, LANES = 2, 16, 16
NW = NC * NS
B, HIST, D = 16384, 200, 16
VOCAB = 1000000
SPW = B // NW
CS = 16
NCH = SPW // CS
NI = CS * HIST
GW = 128
NG = NI // GW
TBLK = 131072


def _tc_matvec_body(tbl_ref, w_ref, o_ref):
    o_ref[...] = jnp.dot(w_ref[...], tbl_ref[...],
                         preferred_element_type=jnp.float32)[0]


def _tc_matvec(table_t, w_row):
    return pl.pallas_call(
        _tc_matvec_body,
        grid=(pl.cdiv(VOCAB, TBLK),),
        in_specs=[pl.BlockSpec((D, TBLK), lambda i: (0, i)),
                  pl.BlockSpec((1, D), lambda i: (0, 0))],
        out_specs=pl.BlockSpec((TBLK,), lambda i: (i,)),
        out_shape=jax.ShapeDtypeStruct((VOCAB,), jnp.float32),
    )(table_t, w_row)


def _sc_body(x_hbm, t_hbm, bias_hbm, out_hbm,
             idx_a, idx_b, vals_a, vals_b, bias_v, out_v, sem_idx, sem_g):
    cid = lax.axis_index("c")
    sid = lax.axis_index("s")
    wid = sid * NC + cid
    s0 = wid * SPW

    pltpu.sync_copy(bias_hbm, bias_v)

    idx_bufs = (idx_a, idx_b)
    vals_bufs = (vals_a, vals_b)

    def idx_fire(c, buf):
        ib = idx_bufs[buf]

        def fire(r, carry):
            pltpu.async_copy(x_hbm.at[s0 + c * CS + r, :],
                             ib.at[pl.ds(r * HIST, HIST)], sem_idx)
            return carry

        lax.fori_loop(0, CS, fire, 0)

    def idx_wait(c, buf):
        ib = idx_bufs[buf]

        def drain(r, carry):
            pltpu.make_async_copy(x_hbm.at[s0 + c * CS + r, :],
                                  ib.at[pl.ds(r * HIST, HIST)],
                                  sem_idx).wait()
            return carry

        lax.fori_loop(0, CS, drain, 0)

    def gather_fire(buf):
        ib, vb = idx_bufs[buf], vals_bufs[buf]

        def fire(g, carry):
            pltpu.async_copy(t_hbm.at[ib.at[pl.ds(g * GW, GW)]],
                             vb.at[pl.ds(g * GW, GW)], sem_g)
            return carry

        lax.fori_loop(0, NG, fire, 0)

    def gather_drain(buf):
        pltpu.make_async_copy(t_hbm.at[pl.ds(0, NI)],
                              vals_bufs[buf], sem_g).wait()

    lanes = lax.iota(jnp.int32, LANES)
    himask = lanes >= 8

    def accumulate(c, buf):
        vb = vals_bufs[buf]

        def sample_body(s, qvec):
            base = s * HIST
            vs = [vb[pl.ds(base + 16 * k, 16)] for k in range(12)]
            acc01 = (vs[0] + vs[1]) + (vs[2] + vs[3])
            acc23 = (vs[4] + vs[5]) + (vs[6] + vs[7])
            acc45 = (vs[8] + vs[9]) + (vs[10] + vs[11])
            tail = vb[pl.ds(base + 184, 16)]
            acc = (acc01 + acc23) + (acc45 +
                                     jnp.where(himask, tail, jnp.float32(0)))
            q = jnp.sum(acc) * jnp.float32(1.0 / HIST)
            return jnp.where(lanes == s, q, qvec)

        qvec = lax.fori_loop(0, CS, sample_body,
                             jnp.zeros((LANES,), jnp.float32))
        z = qvec + bias_v[...]
        e = jnp.exp(-jnp.abs(z))
        sp = jnp.float32(1.0) / (jnp.float32(1.0) + e)
        res = jnp.where(z >= 0, sp, jnp.float32(1.0) - sp)
        yi = (res * jnp.float32(1e4) + jnp.float32(0.5)).astype(jnp.int32)
        out_v[pl.ds(c * CS, CS)] = yi.astype(jnp.float32) / jnp.float32(1e4)

    idx_fire(0, 0)
    idx_fire(1, 1)
    idx_wait(0, 0)
    gather_fire(0)

    def subchunk(c, buf):
        gather_drain(buf)
        idx_wait(c + 1, 1 - buf)
        gather_fire(1 - buf)
        idx_fire(c + 2, buf)
        accumulate(c, buf)

    def outer(i, carry):
        c = 2 * i
        subchunk(c, 0)
        subchunk(c + 1, 1)
        return carry

    lax.fori_loop(0, (NCH - 2) // 2, outer, 0)

    c_last = jnp.int32(NCH - 2)
    gather_drain(0)
    idx_wait(jnp.int32(NCH - 1), 1)
    gather_fire(1)
    accumulate(c_last, 0)
    gather_drain(1)
    accumulate(c_last + 1, 1)

    pltpu.sync_copy(out_v, out_hbm.at[pl.ds(wid * SPW, SPW)])


@functools.partial(jax.jit, static_argnames=())
def kernel(x, table, W, b):
    assert x.shape == (B, HIST) and table.shape == (VOCAB, D)
    t = _tc_matvec(table.T, W.astype(jnp.float32).reshape(1, D))
    bias16 = jnp.broadcast_to(b.astype(jnp.float32).reshape(1), (LANES,))
    mesh = plsc.VectorSubcoreMesh(core_axis_name="c", subcore_axis_name="s",
                                  num_cores=NC, num_subcores=NS)
    kfn = pl.kernel(
        _sc_body,
        out_type=jax.ShapeDtypeStruct((B,), jnp.float32),
        mesh=mesh,
        compiler_params=pltpu.CompilerParams(needs_layout_passes=False,
                                             use_tc_tiling_on_sc=False),
        scratch_types=[
            pltpu.VMEM((NI,), jnp.int32),
            pltpu.VMEM((NI,), jnp.int32),
            pltpu.VMEM((NI,), jnp.float32),
            pltpu.VMEM((NI,), jnp.float32),
            pltpu.VMEM((LANES,), jnp.float32),
            pltpu.VMEM((SPW,), jnp.float32),
            pltpu.SemaphoreType.DMA,
            pltpu.SemaphoreType.DMA,
        ],
    )
    out = kfn(x.astype(jnp.int32), t, bias16)
    return out.reshape(B, 1)


# speedup vs baseline: 1.2447x; 1.2447x over previous
import functools

import jax
import jax.numpy as jnp
from jax import lax
from jax.experimental import pallas as pl
from jax.experimental.pallas import tpu as pltpu
from jax.experimental.pallas import tpu_sc as plsc

NC, NS, LANES = 2, 16, 16
NW = NC * NS
B, HIST, D = 16384, 200, 16
VOCAB = 1000000
SPW = B // NW
CS = 128
NCH = SPW // CS
NV = CS // LANES
TBLK = 131072


def _tc_matvec_body(tbl_ref, w_ref, o_ref):
    o_ref[...] = jnp.dot(w_ref[...], tbl_ref[...],
                         preferred_element_type=jnp.float32)[0]


def _tc_matvec(table_t, w_row):
    return pl.pallas_call(
        _tc_matvec_body,
        grid=(pl.cdiv(VOCAB, TBLK),),
        in_specs=[pl.BlockSpec((D, TBLK), lambda i: (0, i)),
                  pl.BlockSpec((1, D), lambda i: (0, 0))],
        out_specs=pl.BlockSpec((TBLK,), lambda i: (i,)),
        out_shape=jax.ShapeDtypeStruct((VOCAB,), jnp.float32),
    )(table_t, w_row)


def _sc_body(x_hbm, t_hbm, bias_hbm, out_hbm,
             idx_a, idx_b, vals_a, vals_b, bias_v, out_v, sem_idx, sem_g):
    cid = lax.axis_index("c")
    sid = lax.axis_index("s")
    wid = sid * NC + cid
    s0 = wid * SPW

    pltpu.sync_copy(bias_hbm, bias_v)

    idx_bufs = (idx_a, idx_b)
    vals_bufs = (vals_a, vals_b)

    def idx_fire(c, buf):
        pltpu.async_copy(x_hbm.at[:, pl.ds(s0 + c * CS, CS)], idx_bufs[buf],
                         sem_idx)

    def idx_wait(c, buf):
        pltpu.make_async_copy(x_hbm.at[:, pl.ds(s0 + c * CS, CS)],
                              idx_bufs[buf], sem_idx).wait()

    def gather_fire(buf):
        ib, vb = idx_bufs[buf], vals_bufs[buf]

        def fire(j, carry):
            pltpu.async_copy(t_hbm.at[ib.at[j, :]], vb.at[j, :], sem_g)
            return carry

        lax.fori_loop(0, HIST, fire, 0)

    def gather_drain(buf):
        ib, vb = idx_bufs[buf], vals_bufs[buf]

        def drain(j, carry):
            pltpu.make_async_copy(t_hbm.at[ib.at[j, :]], vb.at[j, :],
                                  sem_g).wait()
            return carry

        lax.fori_loop(0, HIST, drain, 0)

    def accumulate(c, buf):
        vb = vals_bufs[buf]
        z16 = jnp.zeros((LANES,), jnp.float32)

        def step(j, accs):
            return tuple(accs[v] + vb[j, pl.ds(v * LANES, LANES)]
                         for v in range(NV))

        accs = lax.fori_loop(0, HIST, step, (z16,) * NV)
        for v in range(NV):
            qvec = accs[v] * jnp.float32(1.0 / HIST)
            z = qvec + bias_v[...]
            e = jnp.exp(-jnp.abs(z))
            sp = jnp.float32(1.0) / (jnp.float32(1.0) + e)
            res = jnp.where(z >= 0, sp, jnp.float32(1.0) - sp)
            yi = (res * jnp.float32(1e4) + jnp.float32(0.5)).astype(jnp.int32)
            out_v[pl.ds(c * CS + v * LANES, LANES)] = (
                yi.astype(jnp.float32) / jnp.float32(1e4))

    idx_fire(0, 0)
    idx_fire(1, 1)
    idx_wait(0, 0)
    gather_fire(0)

    def subchunk(c, buf):
        gather_drain(buf)
        idx_wait(c + 1, 1 - buf)
        gather_fire(1 - buf)
        idx_fire(c + 2, buf)
        accumulate(c, buf)

    def outer(i, carry):
        c = 2 * i
        subchunk(c, 0)
        subchunk(c + 1, 1)
        return carry

    lax.fori_loop(0, (NCH - 2) // 2, outer, 0)

    c_last = jnp.int32(NCH - 2)
    gather_drain(0)
    idx_wait(jnp.int32(NCH - 1), 1)
    gather_fire(1)
    accumulate(c_last, 0)
    gather_drain(1)
    accumulate(c_last + 1, 1)

    pltpu.sync_copy(out_v, out_hbm.at[pl.ds(wid * SPW, SPW)])


@functools.partial(jax.jit, static_argnames=())
def kernel(x, table, W, b):
    assert x.shape == (B, HIST) and table.shape == (VOCAB, D)
    t = _tc_matvec(table.T, W.astype(jnp.float32).reshape(1, D))
    bias16 = jnp.broadcast_to(b.astype(jnp.float32).reshape(1), (LANES,))
    mesh = plsc.VectorSubcoreMesh(core_axis_name="c", subcore_axis_name="s",
                                  num_cores=NC, num_subcores=NS)
    kfn = pl.kernel(
        _sc_body,
        out_type=jax.ShapeDtypeStruct((B,), jnp.float32),
        mesh=mesh,
        compiler_params=pltpu.CompilerParams(needs_layout_passes=False,
                                             use_tc_tiling_on_sc=False),
        scratch_types=[
            pltpu.VMEM((HIST, CS), jnp.int32),
            pltpu.VMEM((HIST, CS), jnp.int32),
            pltpu.VMEM((HIST, CS), jnp.float32),
            pltpu.VMEM((HIST, CS), jnp.float32),
            pltpu.VMEM((LANES,), jnp.float32),
            pltpu.VMEM((SPW,), jnp.float32),
            pltpu.SemaphoreType.DMA,
            pltpu.SemaphoreType.DMA,
        ],
    )
    out = kfn(x.astype(jnp.int32).T, t, bias16)
    return out.reshape(B, 1)


# speedup vs baseline: 2.2960x; 1.8446x over previous
import functools

import jax
import jax.numpy as jnp
from jax import lax
from jax.experimental import pallas as pl
from jax.experimental.pallas import tpu as pltpu
from jax.experimental.pallas import tpu_sc as plsc

NC, NS, LANES = 2, 16, 16
NW = NC * NS
B, HIST, D = 16384, 200, 16
VOCAB = 1000000
SPW = B // NW
CS = 64
NCH = SPW // CS
NV = CS // LANES
TBLK = 131072


def _tc_matvec_body(tbl_ref, w_ref, o_ref):
    o_ref[...] = jnp.dot(w_ref[...], tbl_ref[...],
                         preferred_element_type=jnp.float32)[0]


def _tc_matvec(table_t, w_row):
    return pl.pallas_call(
        _tc_matvec_body,
        grid=(pl.cdiv(VOCAB, TBLK),),
        in_specs=[pl.BlockSpec((D, TBLK), lambda i: (0, i)),
                  pl.BlockSpec((1, D), lambda i: (0, 0))],
        out_specs=pl.BlockSpec((TBLK,), lambda i: (i,)),
        out_shape=jax.ShapeDtypeStruct((VOCAB,), jnp.float32),
    )(table_t, w_row)


def _sc_body(x_hbm, t_hbm, bias_hbm, out_hbm,
             idx_a, idx_b, vals_a, vals_b, bias_v, out_v, t_sh,
             sem_idx, sem_g):
    cid = lax.axis_index("c")
    sid = lax.axis_index("s")
    wid = sid * NC + cid
    s0 = wid * SPW

    pltpu.sync_copy(bias_hbm, bias_v)

    @pl.when(sid == 0)
    def _():
        pltpu.sync_copy(t_hbm, t_sh)

    plsc.subcore_barrier()

    idx_bufs = (idx_a, idx_b)
    vals_bufs = (vals_a, vals_b)

    def idx_fire(c, buf):
        pltpu.async_copy(x_hbm.at[:, pl.ds(s0 + c * CS, CS)], idx_bufs[buf],
                         sem_idx)

    def idx_wait(c, buf):
        pltpu.make_async_copy(x_hbm.at[:, pl.ds(s0 + c * CS, CS)],
                              idx_bufs[buf], sem_idx).wait()

    def gather_fire(buf):
        ib, vb = idx_bufs[buf], vals_bufs[buf]

        def fire(j, carry):
            pltpu.async_copy(t_sh.at[ib.at[j, :]], vb.at[j, :], sem_g)
            return carry

        lax.fori_loop(0, HIST, fire, 0)

    def gather_drain(buf):
        ib, vb = idx_bufs[buf], vals_bufs[buf]

        def drain(j, carry):
            pltpu.make_async_copy(t_sh.at[ib.at[j, :]], vb.at[j, :],
                                  sem_g).wait()
            return carry

        lax.fori_loop(0, HIST, drain, 0)

    def accumulate(c, buf):
        vb = vals_bufs[buf]
        z16 = jnp.zeros((LANES,), jnp.float32)

        def step(j, accs):
            return tuple(accs[v] + vb[j, pl.ds(v * LANES, LANES)]
                         for v in range(NV))

        accs = lax.fori_loop(0, HIST, step, (z16,) * NV)
        for v in range(NV):
            qvec = accs[v] * jnp.float32(1.0 / HIST)
            z = qvec + bias_v[...]
            e = jnp.exp(-jnp.abs(z))
            sp = jnp.float32(1.0) / (jnp.float32(1.0) + e)
            res = jnp.where(z >= 0, sp, jnp.float32(1.0) - sp)
            yi = (res * jnp.float32(1e4) + jnp.float32(0.5)).astype(jnp.int32)
            out_v[pl.ds(c * CS + v * LANES, LANES)] = (
                yi.astype(jnp.float32) / jnp.float32(1e4))

    idx_fire(0, 0)
    idx_fire(1, 1)
    idx_wait(0, 0)
    gather_fire(0)

    def subchunk(c, buf):
        gather_drain(buf)
        idx_wait(c + 1, 1 - buf)
        gather_fire(1 - buf)
        idx_fire(c + 2, buf)
        accumulate(c, buf)

    def outer(i, carry):
        c = 2 * i
        subchunk(c, 0)
        subchunk(c + 1, 1)
        return carry

    lax.fori_loop(0, (NCH - 2) // 2, outer, 0)

    c_last = jnp.int32(NCH - 2)
    gather_drain(0)
    idx_wait(jnp.int32(NCH - 1), 1)
    gather_fire(1)
    accumulate(c_last, 0)
    gather_drain(1)
    accumulate(c_last + 1, 1)

    pltpu.sync_copy(out_v, out_hbm.at[pl.ds(wid * SPW, SPW)])


@functools.partial(jax.jit, static_argnames=())
def kernel(x, table, W, b):
    assert x.shape == (B, HIST) and table.shape == (VOCAB, D)
    t = _tc_matvec(table.T, W.astype(jnp.float32).reshape(1, D))
    bias16 = jnp.broadcast_to(b.astype(jnp.float32).reshape(1), (LANES,))
    mesh = plsc.VectorSubcoreMesh(core_axis_name="c", subcore_axis_name="s",
                                  num_cores=NC, num_subcores=NS)
    kfn = pl.kernel(
        _sc_body,
        out_type=jax.ShapeDtypeStruct((B,), jnp.float32),
        mesh=mesh,
        compiler_params=pltpu.CompilerParams(needs_layout_passes=False,
                                             use_tc_tiling_on_sc=False),
        scratch_types=[
            pltpu.VMEM((HIST, CS), jnp.int32),
            pltpu.VMEM((HIST, CS), jnp.int32),
            pltpu.VMEM((HIST, CS), jnp.float32),
            pltpu.VMEM((HIST, CS), jnp.float32),
            pltpu.VMEM((LANES,), jnp.float32),
            pltpu.VMEM((SPW,), jnp.float32),
            pltpu.VMEM_SHARED((VOCAB,), jnp.float32),
            pltpu.SemaphoreType.DMA,
            pltpu.SemaphoreType.DMA,
        ],
    )
    out = kfn(x.astype(jnp.int32).T, t, bias16)
    return out.reshape(B, 1)


# speedup vs baseline: 2.3042x; 1.0036x over previous
import functools

import jax
import jax.numpy as jnp
from jax import lax
from jax.experimental import pallas as pl
from jax.experimental.pallas import tpu as pltpu
from jax.experimental.pallas import tpu_sc as plsc

NC, NS, LANES = 2, 16, 16
NW = NC * NS
B, HIST, D = 16384, 200, 16
VOCAB = 1000000
SPW = B // NW
CS = 64
NCH = SPW // CS
NV = CS // LANES
TBLK = 262144


def _tc_matvec_body(tbl_ref, w_ref, o_ref):
    o_ref[...] = jnp.dot(w_ref[...], tbl_ref[...],
                         preferred_element_type=jnp.float32)[0]


def _tc_matvec(table_t, w_row):
    return pl.pallas_call(
        _tc_matvec_body,
        grid=(pl.cdiv(VOCAB, TBLK),),
        in_specs=[pl.BlockSpec((D, TBLK), lambda i: (0, i)),
                  pl.BlockSpec((1, D), lambda i: (0, 0))],
        out_specs=pl.BlockSpec((TBLK,), lambda i: (i,)),
        out_shape=jax.ShapeDtypeStruct((VOCAB,), jnp.float32),
    )(table_t, w_row)


def _sc_body(x_hbm, t_hbm, bias_hbm, out_hbm,
             idx_a, idx_b, vals_a, vals_b, bias_v, out_v, t_sh,
             sem_idx, sem_g):
    cid = lax.axis_index("c")
    sid = lax.axis_index("s")
    wid = sid * NC + cid
    s0 = wid * SPW

    pltpu.sync_copy(bias_hbm, bias_v)

    idx_bufs = (idx_a, idx_b)
    vals_bufs = (vals_a, vals_b)

    def idx_fire(c, buf):
        pltpu.async_copy(x_hbm.at[:, pl.ds(s0 + c * CS, CS)], idx_bufs[buf],
                         sem_idx)

    def idx_wait(c, buf):
        pltpu.make_async_copy(x_hbm.at[:, pl.ds(s0 + c * CS, CS)],
                              idx_bufs[buf], sem_idx).wait()

    def gather_fire(buf):
        ib, vb = idx_bufs[buf], vals_bufs[buf]

        def fire(j, carry):
            pltpu.async_copy(t_sh.at[ib.at[j, :]], vb.at[j, :], sem_g)
            return carry

        lax.fori_loop(0, HIST, fire, 0)

    def gather_drain(buf):
        ib, vb = idx_bufs[buf], vals_bufs[buf]

        def drain(j, carry):
            pltpu.make_async_copy(t_sh.at[ib.at[j, :]], vb.at[j, :],
                                  sem_g).wait()
            return carry

        lax.fori_loop(0, HIST, drain, 0)

    def accumulate(c, buf):
        vb = vals_bufs[buf]
        z16 = jnp.zeros((LANES,), jnp.float32)

        def step(j, accs):
            return tuple(accs[v] + vb[j, pl.ds(v * LANES, LANES)]
                         for v in range(NV))

        accs = lax.fori_loop(0, HIST, step, (z16,) * NV)
        for v in range(NV):
            qvec = accs[v] * jnp.float32(1.0 / HIST)
            z = qvec + bias_v[...]
            e = jnp.exp(-jnp.abs(z))
            sp = jnp.float32(1.0) / (jnp.float32(1.0) + e)
            res = jnp.where(z >= 0, sp, jnp.float32(1.0) - sp)
            yi = (res * jnp.float32(1e4) + jnp.float32(0.5)).astype(jnp.int32)
            out_v[pl.ds(c * CS + v * LANES, LANES)] = (
                yi.astype(jnp.float32) / jnp.float32(1e4))

    idx_fire(0, 0)
    idx_fire(1, 1)

    @pl.when(sid == 0)
    def _():
        pltpu.sync_copy(t_hbm, t_sh)

    plsc.subcore_barrier()

    idx_wait(0, 0)
    gather_fire(0)

    def subchunk(c, buf):
        gather_drain(buf)
        idx_wait(c + 1, 1 - buf)
        gather_fire(1 - buf)
        idx_fire(c + 2, buf)
        accumulate(c, buf)

    def outer(i, carry):
        c = 2 * i
        subchunk(c, 0)
        subchunk(c + 1, 1)
        return carry

    lax.fori_loop(0, (NCH - 2) // 2, outer, 0)

    c_last = jnp.int32(NCH - 2)
    gather_drain(0)
    idx_wait(jnp.int32(NCH - 1), 1)
    gather_fire(1)
    accumulate(c_last, 0)
    gather_drain(1)
    accumulate(c_last + 1, 1)

    pltpu.sync_copy(out_v, out_hbm.at[pl.ds(wid * SPW, SPW)])


@functools.partial(jax.jit, static_argnames=())
def kernel(x, table, W, b):
    assert x.shape == (B, HIST) and table.shape == (VOCAB, D)
    t = _tc_matvec(table.T, W.astype(jnp.float32).reshape(1, D))
    bias16 = jnp.broadcast_to(b.astype(jnp.float32).reshape(1), (LANES,))
    mesh = plsc.VectorSubcoreMesh(core_axis_name="c", subcore_axis_name="s",
                                  num_cores=NC, num_subcores=NS)
    kfn = pl.kernel(
        _sc_body,
        out_type=jax.ShapeDtypeStruct((B,), jnp.float32),
        mesh=mesh,
        compiler_params=pltpu.CompilerParams(needs_layout_passes=False,
                                             use_tc_tiling_on_sc=False),
        scratch_types=[
            pltpu.VMEM((HIST, CS), jnp.int32),
            pltpu.VMEM((HIST, CS), jnp.int32),
            pltpu.VMEM((HIST, CS), jnp.float32),
            pltpu.VMEM((HIST, CS), jnp.float32),
            pltpu.VMEM((LANES,), jnp.float32),
            pltpu.VMEM((SPW,), jnp.float32),
            pltpu.VMEM_SHARED((VOCAB,), jnp.float32),
            pltpu.SemaphoreType.DMA,
            pltpu.SemaphoreType.DMA,
        ],
    )
    out = kfn(x.astype(jnp.int32).T, t, bias16)
    return out.reshape(B, 1)
